# Initial kernel scaffold; baseline (speedup 1.0000x reference)
import functools

import jax
import jax.numpy as jnp
from jax import lax
from jax.experimental import pallas as pl
from jax.experimental.pallas import tpu as pltpu
from jax.experimental.pallas import tpu_sc as plsc

NC = 2
NS = 16
L = 16
NW = NC * NS
K = 128


def _ceil_to(a, m):
    return (a + m - 1) // m * m


def _deg_body(B, RPT, de_hbm, degp_hbm, didx_v, val_v, zb_v, deg_sp):
    c = lax.axis_index("c")
    s = lax.axis_index("s")
    w = s * NC + c

    def fill_val(i, _):
        val_v[i, :] = jnp.full((L,), 1.0, jnp.float32)
        return 0

    lax.fori_loop(0, K, fill_val, 0)

    def fill_zero(i, _):
        zb_v[i, :] = jnp.zeros((L,), jnp.float32)
        return 0

    lax.fori_loop(0, RPT, fill_zero, 0)
    pltpu.sync_copy(zb_v, deg_sp.at[pl.ds(s * RPT, RPT)])
    plsc.subcore_barrier()

    def body(b, _):
        base = pl.multiple_of(w * (B * K) + b * K, K)
        pltpu.sync_copy(de_hbm.at[pl.ds(base, K)], didx_v)
        pltpu.sync_copy(val_v, deg_sp.at[didx_v], add=True)
        return 0

    lax.fori_loop(0, B, body, 0)
    plsc.subcore_barrier()
    pltpu.sync_copy(deg_sp.at[pl.ds(s * RPT, RPT)],
                    degp_hbm.at[c, pl.ds(s * RPT, RPT)])


def _scat_body(B, RPT, D, g_hbm, se_hbm, de_hbm, outp_hbm,
               sidx_v, didx_v, rows_v, out_sp, sem):
    c = lax.axis_index("c")
    s = lax.axis_index("s")
    w = s * NC + c

    def zr(i, _):
        for j in range(D // L):
            rows_v[i, pl.ds(j * L, L)] = jnp.zeros((L,), jnp.float32)
        return 0

    lax.fori_loop(0, K, zr, 0)
    for t in range(RPT // K):
        pltpu.sync_copy(rows_v, out_sp.at[pl.ds(s * RPT + t * K, K)])
    plsc.subcore_barrier()

    def body(b, _):
        base = pl.multiple_of(w * (B * K) + b * K, K)
        pltpu.sync_copy(se_hbm.at[pl.ds(base, K)], sidx_v)
        pltpu.sync_copy(de_hbm.at[pl.ds(base, K)], didx_v)
        pltpu.async_copy(g_hbm.at[sidx_v], rows_v, sem).wait()
        pltpu.sync_copy(rows_v, out_sp.at[didx_v], add=True)
        return 0

    lax.fori_loop(0, B, body, 0)
    plsc.subcore_barrier()
    pltpu.sync_copy(out_sp.at[pl.ds(s * RPT, RPT)],
                    outp_hbm.at[c, pl.ds(s * RPT, RPT)])


def _scale_body(degp, x, W, g_out):
    deg = degp[0, :, 0:1] + degp[1, :, 0:1] + 1.0
    dinv = lax.rsqrt(deg)
    h = jnp.dot(x[...], W[...], preferred_element_type=jnp.float32)
    g_out[...] = h * dinv


def _fuse_body(degp, p, g, b, W, g2_out):
    deg = degp[0, :, 0:1] + degp[1, :, 0:1] + 1.0
    dinv = lax.rsqrt(deg)
    y = dinv * (p[0] + p[1] + g[...]) + b[...]
    a = jnp.maximum(y, 0.0)
    g2_out[...] = dinv * jnp.dot(a, W[...], preferred_element_type=jnp.float32)


def _finish_body(degp, q, g2, b, x, out):
    deg = degp[0, :, 0:1] + degp[1, :, 0:1] + 1.0
    dinv = lax.rsqrt(deg)
    y = dinv * (q[0] + q[1] + g2[...]) + b[...]
    out[...] = jnp.maximum(y, 0.0) + x[...]


@jax.jit
def kernel(x, edge_index, W1, b1, W2, b2):
    if x.ndim == 3:
        x = jnp.squeeze(x, axis=1)
    N, D = x.shape
    E = edge_index.shape[1]

    RPAD = _ceil_to(N, NS * K)
    RPT = RPAD // NS
    EPW = _ceil_to(-(-E // NW), K)
    B = EPW // K
    E_pad = EPW * NW
    pad_row = jnp.int32(RPAD - 1)

    se = jnp.concatenate(
        [edge_index[0], jnp.full((E_pad - E,), pad_row, jnp.int32)])
    de = jnp.concatenate(
        [edge_index[1], jnp.full((E_pad - E,), pad_row, jnp.int32)])
    xp = jnp.pad(x, ((0, RPAD - N), (0, 0)))

    mesh = plsc.VectorSubcoreMesh(core_axis_name="c", subcore_axis_name="s")

    deg_call = pl.kernel(
        functools.partial(_deg_body, B, RPT),
        out_type=jax.ShapeDtypeStruct((NC, RPAD, L), jnp.float32),
        mesh=mesh,
        scratch_types=[
            pltpu.VMEM((K,), jnp.int32),
            pltpu.VMEM((K, L), jnp.float32),
            pltpu.VMEM((RPT, L), jnp.float32),
            pltpu.VMEM_SHARED((RPAD, L), jnp.float32),
        ],
    )
    degp = deg_call(de)

    scat_call = pl.kernel(
        functools.partial(_scat_body, B, RPT, D),
        out_type=jax.ShapeDtypeStruct((NC, RPAD, D), jnp.float32),
        mesh=mesh,
        scratch_types=[
            pltpu.VMEM((K,), jnp.int32),
            pltpu.VMEM((K,), jnp.int32),
            pltpu.VMEM((K, D), jnp.float32),
            pltpu.VMEM_SHARED((RPAD, D), jnp.float32),
            pltpu.SemaphoreType.DMA,
        ],
    )

    BR = 256
    grid = (RPAD // BR,)
    degp_spec = pl.BlockSpec((NC, BR, L), lambda i: (0, i, 0))
    row_spec = pl.BlockSpec((BR, D), lambda i: (i, 0))
    p_spec = pl.BlockSpec((NC, BR, D), lambda i: (0, i, 0))
    w_spec = pl.BlockSpec((D, D), lambda i: (0, 0))
    b_spec = pl.BlockSpec((1, D), lambda i: (0, 0))
    rows_out = jax.ShapeDtypeStruct((RPAD, D), jnp.float32)

    g1 = pl.pallas_call(
        _scale_body, grid=grid,
        in_specs=[degp_spec, row_spec, w_spec],
        out_specs=row_spec, out_shape=rows_out,
    )(degp, xp, W1)

    p = scat_call(g1, se, de)

    g2 = pl.pallas_call(
        _fuse_body, grid=grid,
        in_specs=[degp_spec, p_spec, row_spec, b_spec, w_spec],
        out_specs=row_spec, out_shape=rows_out,
    )(degp, p, g1, b1.reshape(1, D), W2)

    q = scat_call(g2, se, de)

    out = pl.pallas_call(
        _finish_body, grid=grid,
        in_specs=[degp_spec, p_spec, row_spec, b_spec, row_spec],
        out_specs=row_spec, out_shape=rows_out,
    )(degp, q, g2, b2.reshape(1, D), xp)

    return out[:N]


# speedup vs baseline: 8.5722x; 8.5722x over previous
import functools

import jax
import jax.numpy as jnp
from jax import lax
from jax.experimental import pallas as pl
from jax.experimental.pallas import tpu as pltpu
from jax.experimental.pallas import tpu_sc as plsc

NC = 2
NS = 16
L = 16
NW = NC * NS
K = 128


def _ceil_to(a, m):
    return (a + m - 1) // m * m


def _deg_body(B, RPAD, de_hbm, degp_hbm, didx_v, dl_v, deg_v, db_v, idr_v,
              deg_sp):
    c = lax.axis_index("c")
    s = lax.axis_index("s")
    w = s * NC + c
    R2 = RPAD // 2
    DR = RPAD // 128

    def zero_rows(ref, nrow, ncol):
        def zr(i, _):
            for j in range(ncol // L):
                ref[i, pl.ds(j * L, L)] = jnp.zeros((L,), jnp.float32)
            return 0
        lax.fori_loop(0, nrow, zr, 0)

    zero_rows(deg_v, DR, 128)
    for g in range(DR // L):
        idr_v[pl.ds(g * L, L)] = lax.iota(jnp.int32, L) + g * L
    @pl.when(s == 0)
    def _():
        pltpu.sync_copy(deg_v, deg_sp)
    plsc.subcore_barrier()

    lane = lax.iota(jnp.int32, L)
    for p in range(2):
        lo = p * R2

        def zf(i, _):
            dl_v[pl.ds(i * L, L)] = jnp.zeros((L,), jnp.float32)
            return 0

        lax.fori_loop(0, R2, zf, 0)

        def blk(b, _):
            base = pl.multiple_of(w * (B * K) + b * K, K)
            pltpu.sync_copy(de_hbm.at[pl.ds(base, K)], didx_v)

            def grp(i, _):
                idx = didx_v[pl.ds(i * L, L)]
                m = (idx >= lo) & (idx < lo + R2)
                fi = jnp.where(m, (idx - lo) * L + lane, R2 * L + lane)
                cur = plsc.load_gather(dl_v, [fi])
                plsc.store_scatter(dl_v, [fi], cur + 1.0)
                return 0

            lax.fori_loop(0, K // L, grp, 0)
            return 0

        lax.fori_loop(0, B, blk, 0)

        def red(rr, _):
            for j in range(8):
                p0 = rr * 128 + j * L
                acc = jnp.zeros((L,), jnp.float32)
                for l in range(L):
                    acc = acc + plsc.load_gather(dl_v, [(p0 + lane) * L + l])
                deg_v[lo // 128 + rr, pl.ds(j * L, L)] = acc
            return 0

        lax.fori_loop(0, R2 // 128, red, 0)

    pltpu.sync_copy(deg_v, deg_sp.at[idr_v], add=True)
    plsc.subcore_barrier()

    @pl.when(s == 0)
    def _():
        pltpu.sync_copy(deg_sp, db_v)
        pltpu.sync_copy(db_v, degp_hbm.at[c])


def _scat_body(B, RPT, D, g_hbm, se_hbm, de_hbm, outp_hbm,
               sidx_v, didx_v, rows_v, out_sp, sem):
    c = lax.axis_index("c")
    s = lax.axis_index("s")
    w = s * NC + c

    def zr(i, _):
        for j in range(D // L):
            rows_v[i, pl.ds(j * L, L)] = jnp.zeros((L,), jnp.float32)
        return 0

    lax.fori_loop(0, K, zr, 0)
    for t in range(RPT // K):
        pltpu.sync_copy(rows_v, out_sp.at[pl.ds(s * RPT + t * K, K)])
    plsc.subcore_barrier()

    def body(b, _):
        base = pl.multiple_of(w * (B * K) + b * K, K)
        pltpu.sync_copy(se_hbm.at[pl.ds(base, K)], sidx_v)
        pltpu.sync_copy(de_hbm.at[pl.ds(base, K)], didx_v)
        pltpu.async_copy(g_hbm.at[sidx_v], rows_v, sem).wait()
        pltpu.sync_copy(rows_v, out_sp.at[didx_v], add=True)
        return 0

    lax.fori_loop(0, B, body, 0)
    plsc.subcore_barrier()
    for t in range(RPT // K):
        r0 = s * RPT + t * K
        pltpu.sync_copy(out_sp.at[pl.ds(r0, K)], rows_v)
        pltpu.sync_copy(rows_v, outp_hbm.at[c, pl.ds(r0, K)])


def _dinv_col(degp_ref, nrow):
    i = pl.program_id(0)
    nr = nrow // 128
    d = (degp_ref[0, pl.ds(nr * i, nr), :]
         + degp_ref[1, pl.ds(nr * i, nr), :] + 1.0)
    dinv = lax.rsqrt(d)
    ri = lax.broadcasted_iota(jnp.int32, (nrow, 128), 0)
    li = lax.broadcasted_iota(jnp.int32, (nrow, 128), 1)
    hi = ri // 128
    o = jnp.broadcast_to(dinv[0:1, :], (nrow, 128))
    for k in range(1, nr):
        o = jnp.where(hi == k, jnp.broadcast_to(dinv[k:k + 1, :],
                                                (nrow, 128)), o)
    sel = li == (ri % 128)
    return jnp.sum(jnp.where(sel, o, 0.0), axis=1, keepdims=True)


def _scale_body(degp, x, W, g_out):
    dinv = _dinv_col(degp, x.shape[0])
    h = jnp.dot(x[...], W[...], preferred_element_type=jnp.float32)
    g_out[...] = h * dinv


def _fuse_body(degp, p, g, b, W, g2_out):
    dinv = _dinv_col(degp, g.shape[0])
    y = dinv * (p[0] + p[1] + g[...]) + b[...]
    a = jnp.maximum(y, 0.0)
    g2_out[...] = dinv * jnp.dot(a, W[...], preferred_element_type=jnp.float32)


def _finish_body(degp, q, g2, b, x, out):
    dinv = _dinv_col(degp, x.shape[0])
    y = dinv * (q[0] + q[1] + g2[...]) + b[...]
    out[...] = jnp.maximum(y, 0.0) + x[...]


@jax.jit
def kernel(x, edge_index, W1, b1, W2, b2):
    if x.ndim == 3:
        x = jnp.squeeze(x, axis=1)
    N, D = x.shape
    E = edge_index.shape[1]

    RPAD = _ceil_to(N, NS * K)
    RPT = RPAD // NS
    DR = RPAD // 128
    EPW = _ceil_to(-(-E // NW), K)
    B = EPW // K
    E_pad = EPW * NW
    pad_row = jnp.int32(RPAD - 1)

    se = jnp.concatenate(
        [edge_index[0], jnp.full((E_pad - E,), pad_row, jnp.int32)])
    de = jnp.concatenate(
        [edge_index[1], jnp.full((E_pad - E,), pad_row, jnp.int32)])
    xp = jnp.pad(x, ((0, RPAD - N), (0, 0)))

    mesh = plsc.VectorSubcoreMesh(core_axis_name="c", subcore_axis_name="s")

    deg_call = pl.kernel(
        functools.partial(_deg_body, B, RPAD),
        out_type=jax.ShapeDtypeStruct((NC, DR, 128), jnp.float32),
        mesh=mesh,
        compiler_params=pltpu.CompilerParams(needs_layout_passes=False),
        scratch_types=[
            pltpu.VMEM((K,), jnp.int32),
            pltpu.VMEM((RPAD // 2 * L + L,), jnp.float32),
            pltpu.VMEM((DR, 128), jnp.float32),
            pltpu.VMEM((DR, 128), jnp.float32),
            pltpu.VMEM((DR,), jnp.int32),
            pltpu.VMEM_SHARED((DR, 128), jnp.float32),
        ],
    )
    degp = deg_call(de)

    scat_call = pl.kernel(
        functools.partial(_scat_body, B, RPT, D),
        out_type=jax.ShapeDtypeStruct((NC, RPAD, D), jnp.float32),
        mesh=mesh,
        scratch_types=[
            pltpu.VMEM((K,), jnp.int32),
            pltpu.VMEM((K,), jnp.int32),
            pltpu.VMEM((K, D), jnp.float32),
            pltpu.VMEM_SHARED((RPAD, D), jnp.float32),
            pltpu.SemaphoreType.DMA,
        ],
    )

    BR = 256
    grid = (RPAD // BR,)
    degp_spec = pl.BlockSpec((NC, DR, 128), lambda i: (0, 0, 0))
    row_spec = pl.BlockSpec((BR, D), lambda i: (i, 0))
    p_spec = pl.BlockSpec((NC, BR, D), lambda i: (0, i, 0))
    w_spec = pl.BlockSpec((D, D), lambda i: (0, 0))
    b_spec = pl.BlockSpec((1, D), lambda i: (0, 0))
    rows_out = jax.ShapeDtypeStruct((RPAD, D), jnp.float32)

    g1 = pl.pallas_call(
        _scale_body, grid=grid,
        in_specs=[degp_spec, row_spec, w_spec],
        out_specs=row_spec, out_shape=rows_out,
    )(degp, xp, W1)

    p = scat_call(g1, se, de)

    g2 = pl.pallas_call(
        _fuse_body, grid=grid,
        in_specs=[degp_spec, p_spec, row_spec, b_spec, w_spec],
        out_specs=row_spec, out_shape=rows_out,
    )(degp, p, g1, b1.reshape(1, D), W2)

    q = scat_call(g2, se, de)

    out = pl.pallas_call(
        _finish_body, grid=grid,
        in_specs=[degp_spec, p_spec, row_spec, b_spec, row_spec],
        out_specs=row_spec, out_shape=rows_out,
    )(degp, q, g2, b2.reshape(1, D), xp)

    return out[:N]


# speedup vs baseline: 12.8722x; 1.5016x over previous
import functools

import jax
import jax.numpy as jnp
from jax import lax
from jax.experimental import pallas as pl
from jax.experimental.pallas import tpu as pltpu
from jax.experimental.pallas import tpu_sc as plsc

NC = 2
NS = 16
L = 16
NW = NC * NS
K = 128


def _ceil_to(a, m):
    return (a + m - 1) // m * m


def _deg_body(B, RPAD, de_hbm, degp_hbm, didx_v, dl_v, deg_v, db_v, idr_v,
              deg_sp, semi):
    c = lax.axis_index("c")
    s = lax.axis_index("s")
    w = s * NC + c
    R2 = RPAD // 2
    DR = RPAD // 128
    ci = pltpu.async_copy(de_hbm.at[w], didx_v, semi)

    def zero_rows(ref, nrow, ncol):
        def zr(i, _):
            for j in range(ncol // L):
                ref[i, pl.ds(j * L, L)] = jnp.zeros((L,), jnp.float32)
            return 0
        lax.fori_loop(0, nrow, zr, 0)

    zero_rows(deg_v, DR, 128)
    for g in range(DR // L):
        idr_v[pl.ds(g * L, L)] = lax.iota(jnp.int32, L) + g * L
    @pl.when(s == 0)
    def _():
        pltpu.sync_copy(deg_v, deg_sp)
    plsc.subcore_barrier()

    lane = lax.iota(jnp.int32, L)
    for p in range(2):
        lo = p * R2

        def zf(i, _):
            dl_v[pl.ds(i * L, L)] = jnp.zeros((L,), jnp.float32)
            return 0

        lax.fori_loop(0, R2, zf, 0)
        if p == 0:
            ci.wait()

        def grp(i, _):
            v = didx_v[i // (K // L), pl.ds((i % (K // L)) * L, L)]
            idx = lax.shift_right_logical(v, 16)
            m = (idx >= lo) & (idx < lo + R2)
            fi = jnp.where(m, (idx - lo) * L + lane, R2 * L + lane)
            cur = plsc.load_gather(dl_v, [fi])
            plsc.store_scatter(dl_v, [fi], cur + 1.0)
            return 0

        lax.fori_loop(0, B * (K // L), grp, 0)

        def red(rr, _):
            for j in range(8):
                p0 = rr * 128 + j * L
                acc = jnp.zeros((L,), jnp.float32)
                for l in range(L):
                    acc = acc + plsc.load_gather(dl_v, [(p0 + lane) * L + l])
                deg_v[lo // 128 + rr, pl.ds(j * L, L)] = acc
            return 0

        lax.fori_loop(0, R2 // 128, red, 0)

    pltpu.sync_copy(deg_v, deg_sp.at[idr_v], add=True)
    plsc.subcore_barrier()

    @pl.when(s == 0)
    def _():
        pltpu.sync_copy(deg_sp, db_v)
        pltpu.sync_copy(db_v, degp_hbm.at[c])


def _scat_body(B, RPT, D, g_hbm, ep_hbm, outp_hbm,
               eidx_v, rows0_v, rows1_v, sidx0_v, sidx1_v, didx0_v, didx1_v,
               out_sp, semi, sem0, sem1):
    c = lax.axis_index("c")
    s = lax.axis_index("s")
    w = s * NC + c

    ci = pltpu.async_copy(ep_hbm.at[w], eidx_v, semi)

    def zr(i, _):
        for j in range(D // L):
            rows0_v[i, pl.ds(j * L, L)] = jnp.zeros((L,), jnp.float32)
        return 0

    lax.fori_loop(0, K, zr, 0)
    for t in range(RPT // K):
        pltpu.sync_copy(rows0_v, out_sp.at[pl.ds(s * RPT + t * K, K)])
    ci.wait()
    plsc.subcore_barrier()

    def unpack(b, sidx, didx):
        for j in range(K // L):
            v = eidx_v[b, pl.ds(j * L, L)]
            sidx[pl.ds(j * L, L)] = v & 0xFFFF
            didx[pl.ds(j * L, L)] = lax.shift_right_logical(v, 16)

    def fire(b, sidx, didx, rows, sem):
        unpack(b, sidx, didx)
        pltpu.async_copy(g_hbm.at[sidx], rows, sem)

    def wait(rows, sem):
        pltpu.make_async_copy(g_hbm.at[sidx0_v], rows, sem).wait()

    def scat(didx, rows):
        pltpu.sync_copy(rows, out_sp.at[didx], add=True)

    fire(0, sidx0_v, didx0_v, rows0_v, sem0)
    nfull = (B - 1) // 2

    def body(sb, _):
        fire(2 * sb + 1, sidx1_v, didx1_v, rows1_v, sem1)
        wait(rows0_v, sem0)
        scat(didx0_v, rows0_v)
        fire(2 * sb + 2, sidx0_v, didx0_v, rows0_v, sem0)
        wait(rows1_v, sem1)
        scat(didx1_v, rows1_v)
        return 0

    lax.fori_loop(0, nfull, body, 0)
    if B % 2 == 1:
        wait(rows0_v, sem0)
        scat(didx0_v, rows0_v)
    else:
        fire(B - 1, sidx1_v, didx1_v, rows1_v, sem1)
        wait(rows0_v, sem0)
        scat(didx0_v, rows0_v)
        wait(rows1_v, sem1)
        scat(didx1_v, rows1_v)

    plsc.subcore_barrier()
    for t in range(RPT // K):
        r0 = s * RPT + t * K
        pltpu.sync_copy(out_sp.at[pl.ds(r0, K)], rows0_v)
        pltpu.sync_copy(rows0_v, outp_hbm.at[c, pl.ds(r0, K)])


def _dinv_col(degp_ref, nrow):
    i = pl.program_id(0)
    nr = nrow // 128
    d = (degp_ref[0, pl.ds(nr * i, nr), :]
         + degp_ref[1, pl.ds(nr * i, nr), :] + 1.0)
    dinv = lax.rsqrt(d)
    ri = lax.broadcasted_iota(jnp.int32, (nrow, 128), 0)
    li = lax.broadcasted_iota(jnp.int32, (nrow, 128), 1)
    hi = ri // 128
    o = jnp.broadcast_to(dinv[0:1, :], (nrow, 128))
    for k in range(1, nr):
        o = jnp.where(hi == k, jnp.broadcast_to(dinv[k:k + 1, :],
                                                (nrow, 128)), o)
    sel = li == (ri % 128)
    return jnp.sum(jnp.where(sel, o, 0.0), axis=1, keepdims=True)


def _scale_body(degp, x, W, g_out):
    dinv = _dinv_col(degp, x.shape[0])
    h = jnp.dot(x[...], W[...], preferred_element_type=jnp.float32)
    g_out[...] = h * dinv


def _fuse_body(degp, p, g, b, W, g2_out):
    dinv = _dinv_col(degp, g.shape[0])
    y = dinv * (p[0] + p[1] + g[...]) + b[...]
    a = jnp.maximum(y, 0.0)
    g2_out[...] = dinv * jnp.dot(a, W[...], preferred_element_type=jnp.float32)


def _finish_body(degp, q, g2, b, x, out):
    dinv = _dinv_col(degp, x.shape[0])
    y = dinv * (q[0] + q[1] + g2[...]) + b[...]
    out[...] = jnp.maximum(y, 0.0) + x[...]


@jax.jit
def kernel(x, edge_index, W1, b1, W2, b2):
    if x.ndim == 3:
        x = jnp.squeeze(x, axis=1)
    N, D = x.shape
    E = edge_index.shape[1]

    RPAD = _ceil_to(N, NS * K)
    RPT = RPAD // NS
    DR = RPAD // 128
    EPW = _ceil_to(-(-E // NW), K)
    B = EPW // K
    E_pad = EPW * NW
    pad_row = jnp.int32(RPAD - 1)

    se = jnp.concatenate(
        [edge_index[0], jnp.full((E_pad - E,), pad_row, jnp.int32)])
    de = jnp.concatenate(
        [edge_index[1], jnp.full((E_pad - E,), pad_row, jnp.int32)])
    ep = jnp.bitwise_or(jnp.left_shift(de, 16), se).reshape(NW, B, K)
    xp = jnp.pad(x, ((0, RPAD - N), (0, 0)))

    mesh = plsc.VectorSubcoreMesh(core_axis_name="c", subcore_axis_name="s")

    deg_call = pl.kernel(
        functools.partial(_deg_body, B, RPAD),
        out_type=jax.ShapeDtypeStruct((NC, DR, 128), jnp.float32),
        mesh=mesh,
        compiler_params=pltpu.CompilerParams(needs_layout_passes=False),
        scratch_types=[
            pltpu.VMEM((B, K), jnp.int32),
            pltpu.VMEM((RPAD // 2 * L + L,), jnp.float32),
            pltpu.VMEM((DR, 128), jnp.float32),
            pltpu.VMEM((DR, 128), jnp.float32),
            pltpu.VMEM((DR,), jnp.int32),
            pltpu.VMEM_SHARED((DR, 128), jnp.float32),
            pltpu.SemaphoreType.DMA,
        ],
    )
    degp = deg_call(ep)

    scat_call = pl.kernel(
        functools.partial(_scat_body, B, RPT, D),
        out_type=jax.ShapeDtypeStruct((NC, RPAD, D), jnp.float32),
        mesh=mesh,
        scratch_types=[
            pltpu.VMEM((B, K), jnp.int32),
            pltpu.VMEM((K, D), jnp.float32),
            pltpu.VMEM((K, D), jnp.float32),
            pltpu.VMEM((K,), jnp.int32),
            pltpu.VMEM((K,), jnp.int32),
            pltpu.VMEM((K,), jnp.int32),
            pltpu.VMEM((K,), jnp.int32),
            pltpu.VMEM_SHARED((RPAD, D), jnp.float32),
            pltpu.SemaphoreType.DMA,
            pltpu.SemaphoreType.DMA,
            pltpu.SemaphoreType.DMA,
        ],
    )

    BR = 256
    grid = (RPAD // BR,)
    degp_spec = pl.BlockSpec((NC, DR, 128), lambda i: (0, 0, 0))
    row_spec = pl.BlockSpec((BR, D), lambda i: (i, 0))
    p_spec = pl.BlockSpec((NC, BR, D), lambda i: (0, i, 0))
    w_spec = pl.BlockSpec((D, D), lambda i: (0, 0))
    b_spec = pl.BlockSpec((1, D), lambda i: (0, 0))
    rows_out = jax.ShapeDtypeStruct((RPAD, D), jnp.float32)

    g1 = pl.pallas_call(
        _scale_body, grid=grid,
        in_specs=[degp_spec, row_spec, w_spec],
        out_specs=row_spec, out_shape=rows_out,
    )(degp, xp, W1)

    p = scat_call(g1, ep)

    g2 = pl.pallas_call(
        _fuse_body, grid=grid,
        in_specs=[degp_spec, p_spec, row_spec, b_spec, w_spec],
        out_specs=row_spec, out_shape=rows_out,
    )(degp, p, g1, b1.reshape(1, D), W2)

    q = scat_call(g2, ep)

    out = pl.pallas_call(
        _finish_body, grid=grid,
        in_specs=[degp_spec, p_spec, row_spec, b_spec, row_spec],
        out_specs=row_spec, out_shape=rows_out,
    )(degp, q, g2, b2.reshape(1, D), xp)

    return out[:N]


# speedup vs baseline: 13.6251x; 1.0585x over previous
import functools

import jax
import jax.numpy as jnp
from jax import lax
from jax.experimental import pallas as pl
from jax.experimental.pallas import tpu as pltpu
from jax.experimental.pallas import tpu_sc as plsc

NC = 2
NS = 16
L = 16
NW = NC * NS
K = 128


def _ceil_to(a, m):
    return (a + m - 1) // m * m


def _deg_body(B, RPAD, de_hbm, degp_hbm, didx_v, dl_v, deg_v, db_v, idr_v,
              deg_sp, semi):
    c = lax.axis_index("c")
    s = lax.axis_index("s")
    w = s * NC + c
    R2 = RPAD // 2
    DR = RPAD // 128
    ci = pltpu.async_copy(de_hbm.at[w], didx_v, semi)

    def zero_rows(ref, nrow, ncol):
        def zr(i, _):
            for j in range(ncol // L):
                ref[i, pl.ds(j * L, L)] = jnp.zeros((L,), jnp.float32)
            return 0
        lax.fori_loop(0, nrow, zr, 0)

    zero_rows(deg_v, DR, 128)
    for g in range(DR // L):
        idr_v[pl.ds(g * L, L)] = lax.iota(jnp.int32, L) + g * L
    @pl.when(s == 0)
    def _():
        pltpu.sync_copy(deg_v, deg_sp)
    plsc.subcore_barrier()

    lane = lax.iota(jnp.int32, L)
    for p in range(2):
        lo = p * R2

        def zf(i, _):
            for j in range(8):
                dl_v[pl.ds(i * 128 + j * L, L)] = jnp.zeros((L,), jnp.float32)
            return 0

        lax.fori_loop(0, R2 // 8, zf, 0)
        if p == 0:
            ci.wait()

        def grp(i, _):
            v = didx_v[i // (K // L), pl.ds((i % (K // L)) * L, L)]
            idx = lax.shift_right_logical(v, 16)
            m = (idx >= lo) & (idx < lo + R2)
            fi = jnp.where(m, (idx - lo) * L + lane, R2 * L + lane)
            cur = plsc.load_gather(dl_v, [fi])
            plsc.store_scatter(dl_v, [fi], cur + 1.0)
            return 0

        lax.fori_loop(0, B * (K // L), grp, 0)

        def red(rr, _):
            for j in range(8):
                p0 = rr * 128 + j * L
                acc = jnp.zeros((L,), jnp.float32)
                for l in range(L):
                    acc = acc + plsc.load_gather(dl_v, [(p0 + lane) * L + l])
                deg_v[lo // 128 + rr, pl.ds(j * L, L)] = acc
            return 0

        lax.fori_loop(0, R2 // 128, red, 0)

    pltpu.sync_copy(deg_v, deg_sp.at[idr_v], add=True)
    plsc.subcore_barrier()

    @pl.when(s == 0)
    def _():
        pltpu.sync_copy(deg_sp, db_v)
        pltpu.sync_copy(db_v, degp_hbm.at[c])


def _scat_body(B, RPT, D, g_hbm, ep_hbm, outp_hbm,
               eidx_v, rows0_v, rows1_v, sidx0_v, sidx1_v, didx0_v, didx1_v,
               out_sp, semi, sem0, sem1):
    c = lax.axis_index("c")
    s = lax.axis_index("s")
    w = s * NC + c

    ci = pltpu.async_copy(ep_hbm.at[w], eidx_v, semi)

    def zr(i, _):
        for j in range(D // L):
            rows0_v[i, pl.ds(j * L, L)] = jnp.zeros((L,), jnp.float32)
        return 0

    lax.fori_loop(0, K, zr, 0)
    for t in range(RPT // K):
        pltpu.sync_copy(rows0_v, out_sp.at[pl.ds(s * RPT + t * K, K)])
    ci.wait()
    plsc.subcore_barrier()

    def unpack(b, sidx, didx):
        for j in range(K // L):
            v = eidx_v[b, pl.ds(j * L, L)]
            sidx[pl.ds(j * L, L)] = v & 0xFFFF
            didx[pl.ds(j * L, L)] = lax.shift_right_logical(v, 16)

    def fire(b, sidx, didx, rows, sem):
        unpack(b, sidx, didx)
        pltpu.async_copy(g_hbm.at[sidx], rows, sem)

    def wait(rows, sem):
        pltpu.make_async_copy(g_hbm.at[sidx0_v], rows, sem).wait()

    def scat(didx, rows):
        pltpu.sync_copy(rows, out_sp.at[didx], add=True)

    fire(0, sidx0_v, didx0_v, rows0_v, sem0)
    nfull = (B - 1) // 2

    def body(sb, _):
        fire(2 * sb + 1, sidx1_v, didx1_v, rows1_v, sem1)
        wait(rows0_v, sem0)
        scat(didx0_v, rows0_v)
        fire(2 * sb + 2, sidx0_v, didx0_v, rows0_v, sem0)
        wait(rows1_v, sem1)
        scat(didx1_v, rows1_v)
        return 0

    lax.fori_loop(0, nfull, body, 0)
    if B % 2 == 1:
        wait(rows0_v, sem0)
        scat(didx0_v, rows0_v)
    else:
        fire(B - 1, sidx1_v, didx1_v, rows1_v, sem1)
        wait(rows0_v, sem0)
        scat(didx0_v, rows0_v)
        wait(rows1_v, sem1)
        scat(didx1_v, rows1_v)

    plsc.subcore_barrier()
    for t in range(RPT // K):
        r0 = s * RPT + t * K
        pltpu.sync_copy(out_sp.at[pl.ds(r0, K)], rows0_v)
        pltpu.sync_copy(rows0_v, outp_hbm.at[c, pl.ds(r0, K)])


def _dinv_col(degp_ref, nrow):
    i = pl.program_id(0)
    nr = nrow // 128
    d = (degp_ref[0, pl.ds(nr * i, nr), :]
         + degp_ref[1, pl.ds(nr * i, nr), :] + 1.0)
    dinv = lax.rsqrt(d)
    ri = lax.broadcasted_iota(jnp.int32, (nrow, 128), 0)
    li = lax.broadcasted_iota(jnp.int32, (nrow, 128), 1)
    hi = ri // 128
    o = jnp.broadcast_to(dinv[0:1, :], (nrow, 128))
    for k in range(1, nr):
        o = jnp.where(hi == k, jnp.broadcast_to(dinv[k:k + 1, :],
                                                (nrow, 128)), o)
    sel = li == (ri % 128)
    return jnp.sum(jnp.where(sel, o, 0.0), axis=1, keepdims=True)


def _scale_body(degp, x, W, g_out, dinvb_out):
    nrow = x.shape[0]
    dinv = _dinv_col(degp, nrow)
    h = jnp.dot(x[...], W[...], preferred_element_type=jnp.float32)
    g_out[...] = h * dinv
    dinvb_out[...] = jnp.broadcast_to(dinv, (nrow, 128))


def _fuse_body(dinvb, p, g, b, W, g2_out):
    dinv = dinvb[...]
    y = dinv * (p[0] + p[1] + g[...]) + b[...]
    a = jnp.maximum(y, 0.0)
    g2_out[...] = dinv * jnp.dot(a, W[...], preferred_element_type=jnp.float32)


def _finish_body(dinvb, q, g2, b, x, out):
    dinv = dinvb[...]
    y = dinv * (q[0] + q[1] + g2[...]) + b[...]
    out[...] = jnp.maximum(y, 0.0) + x[...]


@jax.jit
def kernel(x, edge_index, W1, b1, W2, b2):
    if x.ndim == 3:
        x = jnp.squeeze(x, axis=1)
    N, D = x.shape
    E = edge_index.shape[1]

    RPAD = _ceil_to(N, NS * K)
    RPT = RPAD // NS
    DR = RPAD // 128
    EPW = _ceil_to(-(-E // NW), K)
    B = EPW // K
    E_pad = EPW * NW
    pad_row = jnp.int32(RPAD - 1)

    se = jnp.concatenate(
        [edge_index[0], jnp.full((E_pad - E,), pad_row, jnp.int32)])
    de = jnp.concatenate(
        [edge_index[1], jnp.full((E_pad - E,), pad_row, jnp.int32)])
    ep = jnp.bitwise_or(jnp.left_shift(de, 16), se).reshape(NW, B, K)
    xp = jnp.pad(x, ((0, RPAD - N), (0, 0)))

    mesh = plsc.VectorSubcoreMesh(core_axis_name="c", subcore_axis_name="s")

    deg_call = pl.kernel(
        functools.partial(_deg_body, B, RPAD),
        out_type=jax.ShapeDtypeStruct((NC, DR, 128), jnp.float32),
        mesh=mesh,
        compiler_params=pltpu.CompilerParams(needs_layout_passes=False),
        scratch_types=[
            pltpu.VMEM((B, K), jnp.int32),
            pltpu.VMEM((RPAD // 2 * L + L,), jnp.float32),
            pltpu.VMEM((DR, 128), jnp.float32),
            pltpu.VMEM((DR, 128), jnp.float32),
            pltpu.VMEM((DR,), jnp.int32),
            pltpu.VMEM_SHARED((DR, 128), jnp.float32),
            pltpu.SemaphoreType.DMA,
        ],
    )
    degp = deg_call(ep)

    scat_call = pl.kernel(
        functools.partial(_scat_body, B, RPT, D),
        out_type=jax.ShapeDtypeStruct((NC, RPAD, D), jnp.float32),
        mesh=mesh,
        scratch_types=[
            pltpu.VMEM((B, K), jnp.int32),
            pltpu.VMEM((K, D), jnp.float32),
            pltpu.VMEM((K, D), jnp.float32),
            pltpu.VMEM((K,), jnp.int32),
            pltpu.VMEM((K,), jnp.int32),
            pltpu.VMEM((K,), jnp.int32),
            pltpu.VMEM((K,), jnp.int32),
            pltpu.VMEM_SHARED((RPAD, D), jnp.float32),
            pltpu.SemaphoreType.DMA,
            pltpu.SemaphoreType.DMA,
            pltpu.SemaphoreType.DMA,
        ],
    )

    BR = 256
    grid = (RPAD // BR,)
    degp_spec = pl.BlockSpec((NC, DR, 128), lambda i: (0, 0, 0))
    row_spec = pl.BlockSpec((BR, D), lambda i: (i, 0))
    p_spec = pl.BlockSpec((NC, BR, D), lambda i: (0, i, 0))
    w_spec = pl.BlockSpec((D, D), lambda i: (0, 0))
    b_spec = pl.BlockSpec((1, D), lambda i: (0, 0))
    rows_out = jax.ShapeDtypeStruct((RPAD, D), jnp.float32)

    g1, dinvb = pl.pallas_call(
        _scale_body, grid=grid,
        in_specs=[degp_spec, row_spec, w_spec],
        out_specs=[row_spec, row_spec], out_shape=[rows_out, rows_out],
    )(degp, xp, W1)

    p = scat_call(g1, ep)

    g2 = pl.pallas_call(
        _fuse_body, grid=grid,
        in_specs=[row_spec, p_spec, row_spec, b_spec, w_spec],
        out_specs=row_spec, out_shape=rows_out,
    )(dinvb, p, g1, b1.reshape(1, D), W2)

    q = scat_call(g2, ep)

    out = pl.pallas_call(
        _finish_body, grid=grid,
        in_specs=[row_spec, p_spec, row_spec, b_spec, row_spec],
        out_specs=row_spec, out_shape=rows_out,
    )(dinvb, q, g2, b2.reshape(1, D), xp)

    return out[:N]
